# baseline (speedup 1.0000x reference)
import functools

import jax
import jax.numpy as jnp
from jax import lax
from jax.experimental import pallas as pl
from jax.experimental.pallas import tpu as pltpu
from jax.experimental.pallas import tpu_sc as plsc

B = 4096
L = 50
EMBED_DIM = 128
NUM_CORES = 2
NUM_SUBCORES = 16
NUM_WORKERS = NUM_CORES * NUM_SUBCORES
B_PER_W = B // NUM_WORKERS
CB = 4
CHUNK_I = CB * L
N_CHUNKS = B_PER_W // CB
NBUF = 4
NOBUF = 2
NV = EMBED_DIM // 16


def _sc_body(idx_hbm, table_hbm, out_hbm, idx_all, rows, outs, gsems, osems):
    wid = lax.axis_index("s") * NUM_CORES + lax.axis_index("c")
    idx_base = wid * (B_PER_W * L)
    out_base = wid * B_PER_W

    pltpu.sync_copy(idx_hbm.at[pl.ds(idx_base, B_PER_W * L)], idx_all)

    def issue(c, b):
        pltpu.async_copy(
            table_hbm.at[idx_all.at[pl.ds(c * CHUNK_I, CHUNK_I)]],
            rows[b], gsems[b])

    for b in range(NBUF):
        issue(b, b)

    def accumulate(rows_v, out_v):
        def example_body(e, carry):
            row0 = e * L
            accs = [jnp.zeros((16,), jnp.float32) for _ in range(NV)]
            for l in range(L):
                for d in range(NV):
                    accs[d] = accs[d] + rows_v[row0 + l, pl.ds(d * 16, 16)]
            for d in range(NV):
                out_v[e, pl.ds(d * 16, 16)] = accs[d]
            return carry

        lax.fori_loop(0, CB, example_body, 0)

    def ring_body(cc, carry):
        for b in range(NBUF):
            c = cc * NBUF + b
            ob = b % NOBUF
            pltpu.make_async_copy(
                table_hbm.at[idx_all.at[pl.ds(0, CHUNK_I)]], rows[b],
                gsems[b]).wait()

            @pl.when(c >= NOBUF)
            def _():
                pltpu.make_async_copy(
                    outs[ob], out_hbm.at[pl.ds(0, CB)], osems[ob]).wait()

            accumulate(rows[b], outs[ob])
            pltpu.async_copy(
                outs[ob], out_hbm.at[pl.ds(out_base + c * CB, CB)], osems[ob])

            @pl.when(c + NBUF < N_CHUNKS)
            def _():
                issue(c + NBUF, b)
        return carry

    lax.fori_loop(0, N_CHUNKS // NBUF, ring_body, 0)
    for ob in range(NOBUF):
        pltpu.make_async_copy(
            outs[ob], out_hbm.at[pl.ds(0, CB)], osems[ob]).wait()


def _body(idx_hbm, table_hbm, out_hbm,
          idx_all, r0, r1, r2, r3, o0, o1,
          g0, g1, g2, g3, s0, s1):
    _sc_body(idx_hbm, table_hbm, out_hbm, idx_all,
             (r0, r1, r2, r3), (o0, o1), (g0, g1, g2, g3), (s0, s1))


@jax.jit
def kernel(indices, table):
    idx_flat = indices.reshape(-1).astype(jnp.int32)
    run = pl.kernel(
        _body,
        out_type=jax.ShapeDtypeStruct((B, EMBED_DIM), jnp.float32),
        mesh=plsc.VectorSubcoreMesh(core_axis_name="c", subcore_axis_name="s"),
        scratch_types=(
            [pltpu.VMEM((B_PER_W * L,), jnp.int32)]
            + [pltpu.VMEM((CHUNK_I, EMBED_DIM), jnp.float32)] * NBUF
            + [pltpu.VMEM((CB, EMBED_DIM), jnp.float32)] * NOBUF
            + [pltpu.SemaphoreType.DMA] * (NBUF + NOBUF)
        ),
    )
    return run(idx_flat, table)


# speedup vs baseline: 1.0557x; 1.0557x over previous
import functools

import jax
import jax.numpy as jnp
from jax import lax
from jax.experimental import pallas as pl
from jax.experimental.pallas import tpu as pltpu
from jax.experimental.pallas import tpu_sc as plsc

B = 4096
L = 50
EMBED_DIM = 128
NUM_CORES = 2
NUM_SUBCORES = 16
NUM_WORKERS = NUM_CORES * NUM_SUBCORES
B_PER_W = B // NUM_WORKERS
CB = 8
CHUNK_I = CB * L
N_CHUNKS = B_PER_W // CB
NBUF = 2
NOBUF = 2
NV = EMBED_DIM // 16


def _sc_body(idx_hbm, table_hbm, out_hbm, idx_all, rows, outs, gsems, osems):
    wid = lax.axis_index("s") * NUM_CORES + lax.axis_index("c")
    idx_base = wid * (B_PER_W * L)
    out_base = wid * B_PER_W

    pltpu.sync_copy(idx_hbm.at[pl.ds(idx_base, B_PER_W * L)], idx_all)

    def issue(c, b):
        pltpu.async_copy(
            table_hbm.at[idx_all.at[pl.ds(c * CHUNK_I, CHUNK_I)]],
            rows[b], gsems[b])

    for b in range(NBUF):
        issue(b, b)

    def accumulate(rows_v, out_v):
        def example_body(e, carry):
            row0 = e * L
            accs = [jnp.zeros((16,), jnp.float32) for _ in range(NV)]
            for l in range(L):
                for d in range(NV):
                    accs[d] = accs[d] + rows_v[row0 + l, pl.ds(d * 16, 16)]
            for d in range(NV):
                out_v[e, pl.ds(d * 16, 16)] = accs[d]
            return carry

        lax.fori_loop(0, CB, example_body, 0)

    def ring_body(cc, carry):
        for b in range(NBUF):
            c = cc * NBUF + b
            ob = b % NOBUF
            pltpu.make_async_copy(
                table_hbm.at[idx_all.at[pl.ds(0, CHUNK_I)]], rows[b],
                gsems[b]).wait()

            @pl.when(c >= NOBUF)
            def _():
                pltpu.make_async_copy(
                    outs[ob], out_hbm.at[pl.ds(0, CB)], osems[ob]).wait()

            accumulate(rows[b], outs[ob])
            pltpu.async_copy(
                outs[ob], out_hbm.at[pl.ds(out_base + c * CB, CB)], osems[ob])

            @pl.when(c + NBUF < N_CHUNKS)
            def _():
                issue(c + NBUF, b)
        return carry

    lax.fori_loop(0, N_CHUNKS // NBUF, ring_body, 0)
    for ob in range(NOBUF):
        pltpu.make_async_copy(
            outs[ob], out_hbm.at[pl.ds(0, CB)], osems[ob]).wait()


def _body(idx_hbm, table_hbm, out_hbm,
          idx_all, r0, r1, o0, o1,
          g0, g1, s0, s1):
    _sc_body(idx_hbm, table_hbm, out_hbm, idx_all,
             (r0, r1), (o0, o1), (g0, g1), (s0, s1))


@jax.jit
def kernel(indices, table):
    idx_flat = indices.reshape(-1).astype(jnp.int32)
    run = pl.kernel(
        _body,
        out_type=jax.ShapeDtypeStruct((B, EMBED_DIM), jnp.float32),
        mesh=plsc.VectorSubcoreMesh(core_axis_name="c", subcore_axis_name="s"),
        scratch_types=(
            [pltpu.VMEM((B_PER_W * L,), jnp.int32)]
            + [pltpu.VMEM((CHUNK_I, EMBED_DIM), jnp.float32)] * NBUF
            + [pltpu.VMEM((CB, EMBED_DIM), jnp.float32)] * NOBUF
            + [pltpu.SemaphoreType.DMA] * (NBUF + NOBUF)
        ),
    )
    return run(idx_flat, table)


# speedup vs baseline: 1.9489x; 1.8461x over previous
import functools

import jax
import jax.numpy as jnp
from jax import lax
from jax.experimental import pallas as pl
from jax.experimental.pallas import tpu as pltpu
from jax.experimental.pallas import tpu_sc as plsc

B = 4096
L = 50
EMBED_DIM = 128
NUM_CORES = 2
NUM_SUBCORES = 16
NUM_WORKERS = NUM_CORES * NUM_SUBCORES
B_PER_W = B // NUM_WORKERS
WINDOW = 8
NV = EMBED_DIM // 16


def _sc_body(idx_hbm, table_hbm, out_hbm, idx_all, acc, sem):
    wid = lax.axis_index("s") * NUM_CORES + lax.axis_index("c")
    out_base = wid * B_PER_W

    pltpu.sync_copy(idx_hbm.at[pl.ds(wid * (B_PER_W * L), B_PER_W * L)],
                    idx_all)

    def zero_body(r, carry):
        for d in range(NV):
            acc[r, pl.ds(d * 16, 16)] = jnp.zeros((16,), jnp.float32)
        return carry

    lax.fori_loop(0, B_PER_W, zero_body, 0)

    def issue(l):
        pltpu.async_copy(table_hbm.at[idx_all.at[pl.ds(l * B_PER_W, B_PER_W)]], acc, sem, add=True)

    def wait_one():
        pltpu.make_async_copy(table_hbm.at[idx_all.at[pl.ds(0, B_PER_W)]], acc, sem).wait()

    def fire_body(l, carry):
        issue(l)

        @pl.when(l >= WINDOW)
        def _():
            wait_one()
        return carry

    lax.fori_loop(0, L, fire_body, 0)

    def drain_body(i, carry):
        wait_one()
        return carry

    lax.fori_loop(0, WINDOW, drain_body, 0)
    pltpu.sync_copy(acc, out_hbm.at[pl.ds(out_base, B_PER_W)])


@jax.jit
def kernel(indices, table):
    idx_t = (indices.astype(jnp.int32)
             .reshape(NUM_WORKERS, B_PER_W, L)
             .transpose(0, 2, 1)
             .reshape(-1))
    run = pl.kernel(
        _sc_body,
        out_type=jax.ShapeDtypeStruct((B, EMBED_DIM), jnp.float32),
        mesh=plsc.VectorSubcoreMesh(core_axis_name="c", subcore_axis_name="s"),
        scratch_types=[
            pltpu.VMEM((L * B_PER_W,), jnp.int32),
            pltpu.VMEM((B_PER_W, EMBED_DIM), jnp.float32),
            pltpu.SemaphoreType.DMA,
        ],
    )
    return run(idx_t, table)
